# Initial kernel scaffold; baseline (speedup 1.0000x reference)
import jax
import jax.numpy as jnp
from jax.experimental import pallas as pl


def kernel(t, atom_types, frac_coords, lattices_rep, num_atoms, node2graph, lattices_mat, atom_emb, W_t, W_edge, b_edge, W_rbf_blocks, W_m_blocks, W_h_blocks, W_e2_blocks, W_o1, W_o2):
    raise NotImplementedError("write your pallas kernel here")


# speedup vs baseline: 10.3765x; 10.3765x over previous
import jax
import jax.numpy as jnp
from jax import lax
from jax.experimental import pallas as pl

B = 128
N_PER = 32
NUM_RADIAL = 128
EMB_ATOM = 128
EMB_EDGE = 128
LATENT = 256
NUM_BLOCKS = 3
CUTOFF = 6.0
NUM_TYPES = 100
P_EXP = 5

GROUP = 1

_WIDTH = CUTOFF / (NUM_RADIAL - 1)


def _silu(x):
    return x * jax.nn.sigmoid(x)


def _graph_energy(types, frac, cell, tvec, atom_emb, W_t, W_edge, b_edge,
                  W_rbf, W_m, W_h, W_e2, W_o1, W_o2):
    n = N_PER
    pos = jnp.dot(frac, cell, preferred_element_type=jnp.float32)
    vec = pos.reshape(1, n, 3) - pos.reshape(n, 1, 3)
    dist = jnp.sqrt(jnp.sum(vec * vec, axis=2, keepdims=True) + 1e-9)
    d = dist * (1.0 / CUTOFF)
    d2 = d * d
    d5 = d2 * d2 * d
    env = 1.0 - 21.0 * d5 + 35.0 * (d5 * d) - 15.0 * (d5 * d2)
    ii = lax.broadcasted_iota(jnp.int32, (n, n, 1), 0)
    jj = lax.broadcasted_iota(jnp.int32, (n, n, 1), 1)
    env = jnp.where((d < 1.0) & (ii != jj), env, 0.0)
    centers = (lax.broadcasted_iota(jnp.int32, (1, NUM_RADIAL), 1)
               .astype(jnp.float32) * _WIDTH).reshape(1, 1, NUM_RADIAL)
    z = (dist - centers) * (1.0 / _WIDTH)
    rbf = jnp.exp(-0.5 * (z * z)) * env
    rbf2 = rbf.reshape(n * n, NUM_RADIAL)

    oh = (types == lax.broadcasted_iota(jnp.int32, (n, NUM_TYPES), 1)
          ).astype(jnp.float32)
    h = (jnp.dot(oh, atom_emb, preferred_element_type=jnp.float32)
         + _silu(jnp.dot(tvec, W_t, preferred_element_type=jnp.float32)))

    hw1 = jnp.dot(h, W_edge[0:EMB_ATOM], preferred_element_type=jnp.float32)
    hw2 = jnp.dot(h, W_edge[EMB_ATOM:2 * EMB_ATOM],
                  preferred_element_type=jnp.float32)
    rw = jnp.dot(rbf2, W_edge[2 * EMB_ATOM:],
                 preferred_element_type=jnp.float32)
    e = _silu(hw1.reshape(n, 1, EMB_EDGE) + hw2.reshape(1, n, EMB_EDGE)
              + rw.reshape(n, n, EMB_EDGE)
              + b_edge.reshape(1, 1, EMB_EDGE)).reshape(n * n, EMB_EDGE)

    for blk in range(NUM_BLOCKS):
        rb = jnp.dot(rbf2, W_rbf[blk], preferred_element_type=jnp.float32)
        m = _silu(jnp.dot(e, W_m[blk], preferred_element_type=jnp.float32)) * rb
        agg = jnp.sum(m.reshape(n, n, EMB_EDGE), axis=0)
        h = h + _silu(jnp.dot(agg, W_h[blk],
                              preferred_element_type=jnp.float32))
        e = e + _silu(jnp.dot(e, W_e2[blk],
                              preferred_element_type=jnp.float32)) * rb

    eps = jnp.dot(_silu(jnp.dot(h, W_o1, preferred_element_type=jnp.float32)),
                  W_o2, preferred_element_type=jnp.float32)
    return jnp.sum(eps, axis=0, keepdims=True)


def _body(types_ref, frac_ref, cell_ref, t_ref, atom_emb_ref, W_t_ref,
          W_edge_ref, b_edge_ref, W_rbf_ref, W_m_ref, W_h_ref, W_e2_ref,
          W_o1_ref, W_o2_ref, out_ref):
    atom_emb = atom_emb_ref[...]
    W_t = W_t_ref[...]
    W_edge = W_edge_ref[...]
    b_edge = b_edge_ref[...]
    W_rbf = W_rbf_ref[...]
    W_m = W_m_ref[...]
    W_h = W_h_ref[...]
    W_e2 = W_e2_ref[...]
    W_o1 = W_o1_ref[...]
    W_o2 = W_o2_ref[...]
    for g in range(GROUP):
        en = _graph_energy(
            types_ref[g], frac_ref[g], cell_ref[g], t_ref[g],
            atom_emb, W_t, W_edge, b_edge, W_rbf, W_m, W_h, W_e2, W_o1, W_o2)
        out_ref[g] = en.reshape(1, 1)


def kernel(t, atom_types, frac_coords, lattices_rep, num_atoms, node2graph,
           lattices_mat, atom_emb, W_t, W_edge, b_edge, W_rbf_blocks,
           W_m_blocks, W_h_blocks, W_e2_blocks, W_o1, W_o2):
    types3 = atom_types.reshape(B, N_PER, 1)
    frac3 = frac_coords.reshape(B, N_PER, 3)
    t3 = t.reshape(B, 1, LATENT)
    b_edge2 = b_edge.reshape(1, EMB_EDGE)

    def per_graph(shape):
        return pl.BlockSpec((GROUP,) + shape[1:],
                            lambda g: (g,) + (0,) * (len(shape) - 1))

    def full(shape):
        return pl.BlockSpec(shape, lambda g: (0,) * len(shape))

    out = pl.pallas_call(
        _body,
        grid=(B // GROUP,),
        in_specs=[
            per_graph((B, N_PER, 1)),
            per_graph((B, N_PER, 3)),
            per_graph((B, 3, 3)),
            per_graph((B, 1, LATENT)),
            full((NUM_TYPES, EMB_ATOM)),
            full((LATENT, EMB_ATOM)),
            full((2 * EMB_ATOM + NUM_RADIAL, EMB_EDGE)),
            full((1, EMB_EDGE)),
            full((NUM_BLOCKS, NUM_RADIAL, EMB_EDGE)),
            full((NUM_BLOCKS, EMB_EDGE, EMB_EDGE)),
            full((NUM_BLOCKS, EMB_EDGE, EMB_ATOM)),
            full((NUM_BLOCKS, EMB_EDGE, EMB_EDGE)),
            full((EMB_ATOM, 64)),
            full((64, 1)),
        ],
        out_specs=pl.BlockSpec((GROUP, 1, 1), lambda g: (g, 0, 0)),
        out_shape=jax.ShapeDtypeStruct((B, 1, 1), jnp.float32),
    )(types3, frac3, lattices_mat, t3, atom_emb, W_t, W_edge, b_edge2,
      W_rbf_blocks, W_m_blocks, W_h_blocks, W_e2_blocks, W_o1, W_o2)
    return out.reshape(B, 1)


# speedup vs baseline: 17.2262x; 1.6601x over previous
import jax
import jax.numpy as jnp
from jax import lax
from jax.experimental import pallas as pl

B = 128
N_PER = 32
NUM_RADIAL = 128
EMB_ATOM = 128
EMB_EDGE = 128
LATENT = 256
NUM_BLOCKS = 3
CUTOFF = 6.0
NUM_TYPES = 100
P_EXP = 5

GROUP = 4

_WIDTH = CUTOFF / (NUM_RADIAL - 1)


def _silu(x):
    return x * (0.5 * jnp.tanh(0.5 * x) + 0.5)


def _body(types_ref, frac_ref, cell_ref, t_ref, atom_emb_ref, W_t_ref,
          W_edge_ref, b_edge_ref, W_rbf_ref, W_m_ref, W_h_ref, W_e2_ref,
          W_o1_ref, W_o2_ref, out_ref):
    n = N_PER
    G = GROUP
    F = EMB_EDGE
    R = NUM_RADIAL

    types = types_ref[0]
    frac = frac_ref[0].reshape(G, n, 3)
    cell = cell_ref[...]
    tmat = t_ref[0]
    W_edge = W_edge_ref[...]

    pos = lax.dot_general(frac, cell, (((2,), (1,)), ((0,), (0,))),
                          preferred_element_type=jnp.float32)
    gram = lax.dot_general(pos, pos, (((2,), (2,)), ((0,), (0,))),
                           preferred_element_type=jnp.float32)
    ii = lax.broadcasted_iota(jnp.int32, (G, n, n), 1)
    jj = lax.broadcasted_iota(jnp.int32, (G, n, n), 2)
    eye = ii == jj
    gdiag = jnp.where(eye, gram, 0.0)
    sq_i = jnp.sum(gdiag, axis=2, keepdims=True)
    sq_j = jnp.sum(gdiag, axis=1, keepdims=True)
    dist2 = jnp.maximum(sq_i + sq_j - 2.0 * gram, 0.0) + 1e-9
    dist = jnp.sqrt(dist2)
    d = dist * (1.0 / CUTOFF)
    d2 = d * d
    d5 = d2 * d2 * d
    env = 1.0 - 21.0 * d5 + 35.0 * (d5 * d) - 15.0 * (d5 * d2)
    env = jnp.where((d < 1.0) & (~eye), env, 0.0)

    centers = (lax.broadcasted_iota(jnp.int32, (1, R), 1)
               .astype(jnp.float32) * _WIDTH).reshape(1, 1, 1, R)
    dist4 = jnp.broadcast_to(dist.reshape(G, n, n, 1), (G, n, n, R))
    env4 = jnp.broadcast_to(env.reshape(G, n, n, 1), (G, n, n, R))
    z = (dist4 - centers) * (1.0 / _WIDTH)
    rbf2 = (jnp.exp(-0.5 * (z * z)) * env4).reshape(G * n * n, R)

    oh = (types == lax.broadcasted_iota(jnp.int32, (G * n, NUM_TYPES), 1)
          ).astype(jnp.float32)
    tw = _silu(jnp.dot(tmat, W_t_ref[...],
                       preferred_element_type=jnp.float32))
    h = (jnp.dot(oh, atom_emb_ref[...], preferred_element_type=jnp.float32)
         .reshape(G, n, EMB_ATOM) + tw.reshape(G, 1, EMB_ATOM)
         ).reshape(G * n, EMB_ATOM)

    hw1 = jnp.dot(h, W_edge[0:EMB_ATOM], preferred_element_type=jnp.float32)
    hw2 = jnp.dot(h, W_edge[EMB_ATOM:2 * EMB_ATOM],
                  preferred_element_type=jnp.float32)
    rw = jnp.dot(rbf2, W_edge[2 * EMB_ATOM:],
                 preferred_element_type=jnp.float32)
    e = _silu(hw1.reshape(G, n, 1, F) + hw2.reshape(G, 1, n, F)
              + rw.reshape(G, n, n, F)
              + b_edge_ref[...].reshape(1, 1, 1, F)).reshape(G * n * n, F)

    for blk in range(NUM_BLOCKS):
        rb = jnp.dot(rbf2, W_rbf_ref[blk], preferred_element_type=jnp.float32)
        m = _silu(jnp.dot(e, W_m_ref[blk],
                          preferred_element_type=jnp.float32)) * rb
        agg = jnp.sum(m.reshape(G, n, n, F), axis=1).reshape(G * n, F)
        h = h + _silu(jnp.dot(agg, W_h_ref[blk],
                              preferred_element_type=jnp.float32))
        e = e + _silu(jnp.dot(e, W_e2_ref[blk],
                              preferred_element_type=jnp.float32)) * rb

    eps = jnp.dot(_silu(jnp.dot(h, W_o1_ref[...],
                                preferred_element_type=jnp.float32)),
                  W_o2_ref[...], preferred_element_type=jnp.float32)
    out_ref[...] = jnp.sum(eps.reshape(G, n, 1), axis=1, keepdims=True)


def kernel(t, atom_types, frac_coords, lattices_rep, num_atoms, node2graph,
           lattices_mat, atom_emb, W_t, W_edge, b_edge, W_rbf_blocks,
           W_m_blocks, W_h_blocks, W_e2_blocks, W_o1, W_o2):
    S = B // GROUP
    types3 = atom_types.reshape(S, GROUP * N_PER, 1)
    frac3 = frac_coords.reshape(S, GROUP * N_PER, 3)
    t3 = t.reshape(S, GROUP, LATENT)
    b_edge2 = b_edge.reshape(1, EMB_EDGE)

    def full(shape):
        return pl.BlockSpec(shape, lambda g: (0,) * len(shape))

    out = pl.pallas_call(
        _body,
        grid=(S,),
        in_specs=[
            pl.BlockSpec((1, GROUP * N_PER, 1), lambda g: (g, 0, 0)),
            pl.BlockSpec((1, GROUP * N_PER, 3), lambda g: (g, 0, 0)),
            pl.BlockSpec((GROUP, 3, 3), lambda g: (g, 0, 0)),
            pl.BlockSpec((1, GROUP, LATENT), lambda g: (g, 0, 0)),
            full((NUM_TYPES, EMB_ATOM)),
            full((LATENT, EMB_ATOM)),
            full((2 * EMB_ATOM + NUM_RADIAL, EMB_EDGE)),
            full((1, EMB_EDGE)),
            full((NUM_BLOCKS, NUM_RADIAL, EMB_EDGE)),
            full((NUM_BLOCKS, EMB_EDGE, EMB_EDGE)),
            full((NUM_BLOCKS, EMB_EDGE, EMB_ATOM)),
            full((NUM_BLOCKS, EMB_EDGE, EMB_EDGE)),
            full((EMB_ATOM, 64)),
            full((64, 1)),
        ],
        out_specs=pl.BlockSpec((GROUP, 1, 1), lambda g: (g, 0, 0)),
        out_shape=jax.ShapeDtypeStruct((B, 1, 1), jnp.float32),
    )(types3, frac3, lattices_mat, t3, atom_emb, W_t, W_edge, b_edge2,
      W_rbf_blocks, W_m_blocks, W_h_blocks, W_e2_blocks, W_o1, W_o2)
    return out.reshape(B, 1)
